# Initial kernel scaffold; baseline (speedup 1.0000x reference)
import jax
import jax.numpy as jnp
from jax.experimental import pallas as pl


def kernel(frame_indices, frame_table, second_table, minute_table, pe):
    raise NotImplementedError("write your pallas kernel here")


# speedup vs baseline: 2.1181x; 2.1181x over previous
import functools

import jax
import jax.numpy as jnp
from jax import lax
from jax.experimental import pallas as pl
from jax.experimental.pallas import tpu as pltpu
from jax.experimental.pallas import tpu_sc as plsc

DIM = 64
MAXF = 432000
B_TOTAL = 1024 * 200
L = 16
NC, NS = 2, 16
NW = NC * NS
SUB = 128
ROWS_PER_W = B_TOTAL // NW
NSUB = ROWS_PER_W // SUB

_INV60 = 1.0 / 60.0
_INV3600 = 1.0 / 3600.0

_mesh = plsc.VectorSubcoreMesh(core_axis_name="c", subcore_axis_name="s")


@functools.partial(
    pl.kernel,
    mesh=_mesh,
    compiler_params=pltpu.CompilerParams(use_tc_tiling_on_sc=False),
    out_type=jax.ShapeDtypeStruct((B_TOTAL, DIM), jnp.float32),
    scratch_types=[
        pltpu.VMEM((NSUB, SUB), jnp.int32),
        pltpu.VMEM((NSUB, SUB), jnp.int32),
        pltpu.VMEM((NSUB, SUB), jnp.int32),
        pltpu.VMEM((SUB, DIM), jnp.float32),
        pltpu.VMEM((SUB, DIM), jnp.float32),
        pltpu.VMEM((SUB, DIM), jnp.float32),
        pltpu.VMEM((SUB, DIM), jnp.float32),
        pltpu.SemaphoreType.DMA,
    ],
)
def _encode(idx_hbm, ftab, stab, mtab, petab, out_hbm,
            idx_v, sidx_v, midx_v, acc, brow, crow, drow, sem):
    wid = lax.axis_index("s") * NC + lax.axis_index("c")
    row0 = wid * ROWS_PER_W

    pltpu.sync_copy(idx_hbm.at[wid], idx_v)

    def derive(j, carry):
        for k in range(SUB // L):
            s = pl.ds(k * L, L)
            f = idx_v[j, s].astype(jnp.float32)
            sidx_v[j, s] = (f * _INV60).astype(jnp.int32)
            midx_v[j, s] = (f * _INV3600).astype(jnp.int32)
        return carry

    lax.fori_loop(0, NSUB, derive, 0)

    def chunk(j, carry):
        cps = [
            pltpu.async_copy(ftab.at[idx_v.at[j]], acc, sem),
            pltpu.async_copy(petab.at[idx_v.at[j]], brow, sem),
            pltpu.async_copy(stab.at[sidx_v.at[j]], crow, sem),
            pltpu.async_copy(mtab.at[midx_v.at[j]], drow, sem),
        ]
        for cp in cps:
            cp.wait()

        def add_row(r, c2):
            for q in range(DIM // L):
                s = pl.ds(q * L, L)
                acc[r, s] = acc[r, s] + brow[r, s] + crow[r, s] + drow[r, s]
            return c2

        lax.fori_loop(0, SUB, add_row, 0)
        pltpu.sync_copy(acc, out_hbm.at[pl.ds(row0 + j * SUB, SUB)])
        return carry

    lax.fori_loop(0, NSUB, chunk, 0)


def kernel(frame_indices, frame_table, second_table, minute_table, pe):
    bsz, seq = frame_indices.shape
    idx = frame_indices.astype(jnp.int32).reshape(NW, NSUB, SUB)
    out = _encode(idx, frame_table, second_table, minute_table, pe)
    return out.reshape(bsz, seq, DIM)
